# baseline (speedup 1.0000x reference)
import functools

import jax
import jax.numpy as jnp
from jax import lax
from jax.experimental import pallas as pl
from jax.experimental.pallas import tpu as pltpu

H = 256
W = 256
G = 4096
GC = 128
NCHUNK = G // GC
ALPHA_THR = 1.0 / 255.0
TRANS_THR = 1e-4
ALPHA_CLAMP = 0.99


def _row_kernel(mx_ref, my_ref, ca_ref, cb_ref, cc_ref, op_ref, col_ref,
                out_ref):
    y = jnp.float32(pl.program_id(0)) + 0.5
    xs = lax.broadcasted_iota(jnp.int32, (W, 1), 0).astype(jnp.float32) + 0.5

    def chunk_body(state):
        k, T, acc = state
        mxk = mx_ref[pl.ds(k, 1), :]
        myk = my_ref[pl.ds(k, 1), :]
        cak = ca_ref[pl.ds(k, 1), :]
        cbk = cb_ref[pl.ds(k, 1), :]
        cck = cc_ref[pl.ds(k, 1), :]
        opk = op_ref[pl.ds(k, 1), :]

        dx = xs - mxk
        dy = y - myk
        sigma = 0.5 * (cak * dx * dx + cck * (dy * dy)) + cbk * dx * dy
        alpha = jnp.minimum(ALPHA_CLAMP, opk * jnp.exp(-sigma))
        valid = (sigma >= 0.0) & (alpha >= ALPHA_THR)
        alpha = jnp.where(valid, alpha, 0.0)
        one_m = 1.0 - alpha

        cp = one_m
        d = 1
        while d < GC:
            shifted = jnp.concatenate(
                [jnp.ones((W, d), jnp.float32), cp[:, :GC - d]], axis=1)
            cp = cp * shifted
            d *= 2
        cp_excl = jnp.concatenate(
            [jnp.ones((W, 1), jnp.float32), cp[:, :GC - 1]], axis=1)

        T_before = T * cp_excl
        T_after = T * cp
        mask = T_after >= TRANS_THR
        contrib = jnp.where(mask, alpha * T_before, 0.0)

        colk = col_ref[pl.ds(k * GC, GC), :]
        acc = acc + jax.lax.dot_general(
            contrib, colk, (((1,), (0,)), ((), ())),
            preferred_element_type=jnp.float32)
        T = T * cp[:, GC - 1:GC]
        return k + 1, T, acc

    def cond(state):
        k, T, _ = state
        return (k < NCHUNK) & (jnp.max(T) >= TRANS_THR)

    init = (jnp.int32(0),
            jnp.ones((W, 1), jnp.float32),
            jnp.zeros((W, 3), jnp.float32))
    _, _, acc = lax.while_loop(cond, chunk_body, init)
    out_ref[0, :, :] = acc


def _render(mx, my, ca, cb, cc, op, col, *, interpret=False):
    grid = (H,)
    gspec = pl.BlockSpec((NCHUNK, GC), lambda i: (0, 0))
    return pl.pallas_call(
        _row_kernel,
        grid=grid,
        in_specs=[gspec, gspec, gspec, gspec, gspec, gspec,
                  pl.BlockSpec((G, 3), lambda i: (0, 0))],
        out_specs=pl.BlockSpec((1, W, 3), lambda i: (i, 0, 0)),
        out_shape=jax.ShapeDtypeStruct((H, W, 3), jnp.float32),
        interpret=interpret,
    )(mx, my, ca, cb, cc, op, col)


@jax.jit
def kernel(means2d, conics, colors, opacities, depths):
    perm = jnp.argsort(depths, stable=True)
    m = means2d[perm]
    c = conics[perm]
    col = colors[perm]
    op = opacities[perm]
    mx = m[:, 0].reshape(NCHUNK, GC)
    my = m[:, 1].reshape(NCHUNK, GC)
    ca = c[:, 0].reshape(NCHUNK, GC)
    cb = c[:, 1].reshape(NCHUNK, GC)
    cc = c[:, 2].reshape(NCHUNK, GC)
    opr = op.reshape(NCHUNK, GC)
    return _render(mx, my, ca, cb, cc, opr, col)
